# baseline (speedup 1.0000x reference)
import functools

import jax
import jax.numpy as jnp
import numpy as np
from jax import lax
from jax.experimental import pallas as pl
from jax.experimental.pallas import tpu as pltpu
from jax.experimental.pallas import tpu_sc as plsc

N = 10000
E = 320000
D = 128
NCLS = 16

SC_CORES = 2
SC_SUBCORES = 16
NTILES = SC_CORES * SC_SUBCORES
EPT = E // NTILES

B = 80
NCHUNK = EPT // B
SB = 25
NSTAGE = NCHUNK // SB
NSLOT = 4
NQUAD = (SB - 1) // NSLOT

BD = 2000
NDCHUNK = EPT // BD
N_PAD = 10240
NSPAN = N_PAD // SC_SUBCORES

_mesh = plsc.VectorSubcoreMesh(
    core_axis_name="c", subcore_axis_name="s",
    num_cores=SC_CORES, num_subcores=SC_SUBCORES)


@functools.partial(
    pl.kernel,
    out_type=jax.ShapeDtypeStruct((SC_CORES * N_PAD,), jnp.float32),
    mesh=_mesh,
    compiler_params=pltpu.CompilerParams(needs_layout_passes=False),
    scratch_types=[
        pltpu.VMEM_SHARED((SC_SUBCORES, N_PAD), jnp.float32),
        pltpu.VMEM((N_PAD,), jnp.float32),
        pltpu.VMEM((SB, B), jnp.int32),
        pltpu.VMEM((BD,), jnp.float32),
        pltpu.VMEM((NSPAN,), jnp.float32),
        pltpu.VMEM((NSPAN,), jnp.float32),
    ],
)
def _deg_kernel(eidx_hbm, w_hbm, out_hbm, parts_sh, deg_v, idx_v, w_v,
                r_acc, r_op):
    cid = lax.axis_index("c")
    sid = lax.axis_index("s")
    wid = cid * SC_SUBCORES + sid

    zero16 = jnp.zeros((16,), jnp.float32)

    def z_body(i, _):
        deg_v[pl.ds(i * 16, 16)] = zero16
        return 0
    lax.fori_loop(0, N_PAD // 16, z_body, 0)

    base = pl.multiple_of(wid * EPT, 8)

    def chunk_body(ci, _):
        off = pl.multiple_of(base + ci * BD, 8)
        pltpu.sync_copy(eidx_hbm.at[1, wid, ci], idx_v)
        pltpu.sync_copy(w_hbm.at[pl.ds(off, BD)], w_v)

        def g_body(r, _):
            for g in range(B // 16):
                idx16 = idx_v[r, pl.ds(g * 16, 16)]
                w16 = w_v[pl.ds(r * B + g * 16, 16)]
                plsc.addupdate_scatter(deg_v, [idx16], w16)
            return 0
        lax.fori_loop(0, SB, g_body, 0)
        return 0
    lax.fori_loop(0, NDCHUNK, chunk_body, 0)

    pltpu.sync_copy(deg_v, parts_sh.at[sid])
    plsc.subcore_barrier()

    span = pl.multiple_of(sid * NSPAN, 8)
    pltpu.sync_copy(parts_sh.at[0, pl.ds(span, NSPAN)], r_acc)

    def red_body(t, _):
        pltpu.sync_copy(parts_sh.at[t + 1, pl.ds(span, NSPAN)], r_op)

        def add_body(i, _):
            sl = pl.ds(i * 16, 16)
            r_acc[sl] = r_acc[sl] + r_op[sl]
            return 0
        lax.fori_loop(0, NSPAN // 16, add_body, 0)
        return 0
    lax.fori_loop(0, SC_SUBCORES - 1, red_body, 0)

    oof = pl.multiple_of(cid * N_PAD + span, 8)
    pltpu.sync_copy(r_acc, out_hbm.at[pl.ds(oof, NSPAN)])


NG = 3
NS = 2


def _make_agg(gdim, sdim):
    @functools.partial(
        pl.kernel,
        out_type=jax.ShapeDtypeStruct((SC_CORES, N, D), jnp.float32),
        mesh=_mesh,
        compiler_params=pltpu.CompilerParams(needs_layout_passes=False,
                                             use_tc_tiling_on_sc=False),
        scratch_types=[
            pltpu.VMEM_SHARED((N, D), jnp.float32),
            pltpu.VMEM((SB, B), jnp.int32),
            pltpu.VMEM((SB, B), jnp.int32),
        ] + [pltpu.VMEM((B, D // 2), jnp.int32)] * NG
          + [pltpu.VMEM((B, D), jnp.float32)] * NS
          + [pltpu.VMEM((B,), jnp.float32)] * NG
          + [pltpu.SemaphoreType.DMA] * (2 * NG + NS),
    )
    def agg(v_hbm, eidx_hbm, w_hbm, out_hbm, acc_sh, gid_l, sid_l, *rest):
        gbufs = rest[:NG]
        sbufs = rest[NG:NG + NS]
        wbufs = rest[NG + NS:2 * NG + NS]
        gsems = rest[2 * NG + NS:3 * NG + NS]
        ssems = rest[3 * NG + NS:3 * NG + 2 * NS]
        wsems = rest[3 * NG + 2 * NS:4 * NG + 2 * NS]

        cid = lax.axis_index("c")
        sid = lax.axis_index("s")
        wid = cid * SC_SUBCORES + sid
        wbase = pl.multiple_of(wid * EPT, 8)

        zero16 = jnp.zeros((16,), jnp.float32)

        def zb_body(i, _):
            for k in range(D // 16):
                sbufs[0][i, pl.ds(k * 16, 16)] = zero16
            return 0
        lax.fori_loop(0, B, zb_body, 0)

        nblk = N // 80

        def z_issue(t, _):
            b = sid + t * SC_SUBCORES

            @pl.when(b < nblk)
            def _():
                pltpu.async_copy(
                    sbufs[0],
                    acc_sh.at[pl.ds(pl.multiple_of(b * 80, 8), 80)],
                    gsems[0])
            return 0
        lax.fori_loop(0, 8, z_issue, 0)

        def z_wait(t, _):
            b = sid + t * SC_SUBCORES

            @pl.when(b < nblk)
            def _():
                pltpu.make_async_copy(
                    sbufs[0], acc_sh.at[pl.ds(0, 80)], gsems[0]).wait()
            return 0
        lax.fori_loop(0, 8, z_wait, 0)
        plsc.subcore_barrier()

        def issue_gather(sb, c, p):
            pltpu.async_copy(v_hbm.at[gid_l.at[c]], gbufs[p], gsems[p])
            woff = pl.multiple_of(wbase + (sb * SB + c) * B, 8)
            pltpu.async_copy(w_hbm.at[pl.ds(woff, B)], wbufs[p], wsems[p])

        def wait_gather(p):
            pltpu.make_async_copy(v_hbm.at[gid_l.at[0]], gbufs[p],
                                  gsems[p]).wait()
            pltpu.make_async_copy(w_hbm.at[pl.ds(0, B)], wbufs[p],
                                  wsems[p]).wait()

        def issue_scatter(c, p):
            pltpu.async_copy(sbufs[p], acc_sh.at[sid_l.at[c]], ssems[p],
                             add=True)

        def wait_scatter(p):
            pltpu.make_async_copy(sbufs[p], acc_sh.at[sid_l.at[0]],
                                  ssems[p]).wait()

        def scale(c, gp, sp):
            gr = gbufs[gp]
            sr = sbufs[sp]
            wv = wbufs[gp]

            def g_body(g, _):
                w16 = wv[pl.ds(g * 16, 16)]
                for j in range(16):
                    e = g * 16 + j
                    wb = jnp.full((16,), w16[j], jnp.float32)
                    for m in range(D // 32):
                        v32 = plsc.bitcast(gr[e, pl.ds(m * 16, 16)],
                                           jnp.bfloat16)
                        va, vb = plsc.unpack(
                            v32, format=plsc.PackFormat.INTERLEAVED)
                        sr[e, pl.ds(m * 32, 16)] = va * wb
                        sr[e, pl.ds(m * 32 + 16, 16)] = vb * wb
                return 0
            lax.fori_loop(0, B // 16, g_body, 0)

        NHEX = (SB - 1) // 6

        def block_body(sb, _):
            pltpu.sync_copy(eidx_hbm.at[gdim, wid, sb], gid_l)
            pltpu.sync_copy(eidx_hbm.at[sdim, wid, sb], sid_l)

            issue_gather(sb, 0, 0)
            issue_gather(sb, 1, 1)

            def hex_body(t, _):
                for j in range(6):
                    c = t * 6 + j
                    gp = j % NG
                    sp = j % NS
                    if j < 2:
                        @pl.when(t > 0)
                        def _():
                            wait_scatter(sp)
                    else:
                        wait_scatter(sp)
                    if j == 5:
                        @pl.when(t < NHEX - 1)
                        def _():
                            issue_gather(sb, c + 2, (j + 2) % NG)
                    else:
                        issue_gather(sb, c + 2, (j + 2) % NG)
                    wait_gather(gp)
                    scale(c, gp, sp)
                    issue_scatter(c, sp)
                return 0
            lax.fori_loop(0, NHEX, hex_body, 0)

            wait_scatter(0)
            wait_gather(0)
            scale(SB - 1, 0, 0)
            issue_scatter(SB - 1, 0)
            wait_scatter(1)
            wait_scatter(0)
            return 0

        lax.fori_loop(0, NSTAGE, block_body, 0)

        plsc.subcore_barrier()

        def o_issue(t, _):
            b = sid + t * SC_SUBCORES

            @pl.when(b < nblk)
            def _():
                ro = pl.multiple_of(b * 80, 8)
                pltpu.async_copy(acc_sh.at[pl.ds(ro, 80)],
                                 out_hbm.at[cid, pl.ds(ro, 80)], gsems[0])
            return 0
        lax.fori_loop(0, 8, o_issue, 0)

        def o_wait(t, _):
            b = sid + t * SC_SUBCORES

            @pl.when(b < nblk)
            def _():
                pltpu.make_async_copy(acc_sh.at[pl.ds(0, 80)],
                                      out_hbm.at[cid, pl.ds(0, 80)],
                                      gsems[0]).wait()
            return 0
        lax.fori_loop(0, 8, o_wait, 0)

    return agg


_agg_fwd = _make_agg(0, 1)
_agg_rev = _make_agg(1, 0)


BN = 2000
GRID = N // BN


def _k1_body(degp_ref, x_ref, w1q_ref, w1_ref, xws_ref, xb_ref, dinv_ref):
    deg = jnp.sum(degp_ref[...], axis=1) + 1.0
    dinv = lax.rsqrt(deg)[:, None]
    x = x_ref[...]
    xwq = jnp.dot(x, w1q_ref[...], preferred_element_type=jnp.float32)
    xwt = jnp.dot(x, w1_ref[...], preferred_element_type=jnp.float32)
    xws_ref[...] = xwq * dinv
    xb_ref[...] = (xwt * dinv).astype(jnp.bfloat16)
    dinv_ref[...] = dinv


def _k1(deg_parts, X, W1q, W1):
    return pl.pallas_call(
        _k1_body,
        grid=(GRID,),
        in_specs=[
            pl.BlockSpec((BN, SC_CORES), lambda i: (i, 0)),
            pl.BlockSpec((BN, D), lambda i: (i, 0)),
            pl.BlockSpec((D, D), lambda i: (0, 0)),
            pl.BlockSpec((D, D), lambda i: (0, 0)),
        ],
        out_specs=[
            pl.BlockSpec((BN, D), lambda i: (i, 0)),
            pl.BlockSpec((BN, D), lambda i: (i, 0)),
            pl.BlockSpec((BN, 1), lambda i: (i, 0)),
        ],
        out_shape=[
            jax.ShapeDtypeStruct((N, D), jnp.float32),
            jax.ShapeDtypeStruct((N, D), jnp.bfloat16),
            jax.ShapeDtypeStruct((N, 1), jnp.float32),
        ],
    )(deg_parts, X, W1q, W1)


def _k2_body(agg_ref, xws_ref, dinv_ref, b_ref, w2a_ref, w2b_ref,
             out_ref, xb_ref):
    a = agg_ref[0] + agg_ref[1] + xws_ref[...]
    h = jax.nn.relu(a * dinv_ref[...] + b_ref[...])
    xwa = jnp.dot(h, w2a_ref[...], preferred_element_type=jnp.float32)
    xwb = jnp.dot(h, w2b_ref[...], preferred_element_type=jnp.float32)
    out_ref[...] = xwa * dinv_ref[...]
    xb_ref[...] = (xwb * dinv_ref[...]).astype(jnp.bfloat16)


def _k2(agg, xws, dinv, b1q, W2A, W2B):
    return pl.pallas_call(
        _k2_body,
        grid=(GRID,),
        in_specs=[
            pl.BlockSpec((SC_CORES, BN, D), lambda i: (0, i, 0)),
            pl.BlockSpec((BN, D), lambda i: (i, 0)),
            pl.BlockSpec((BN, 1), lambda i: (i, 0)),
            pl.BlockSpec((1, D), lambda i: (0, 0)),
            pl.BlockSpec((D, D), lambda i: (0, 0)),
            pl.BlockSpec((D, D), lambda i: (0, 0)),
        ],
        out_specs=[
            pl.BlockSpec((BN, D), lambda i: (i, 0)),
            pl.BlockSpec((BN, D), lambda i: (i, 0)),
        ],
        out_shape=[
            jax.ShapeDtypeStruct((N, D), jnp.float32),
            jax.ShapeDtypeStruct((N, D), jnp.bfloat16),
        ],
    )(agg, xws, dinv, b1q.reshape(1, D), W2A, W2B)


def _k3_body(agg_ref, xws_ref, dinv_ref, b_ref, z_ref, y_ref,
             s_ref, sb_ref, xnew_ref, syo_ref, accx, accy):
    i = pl.program_id(0)
    a = agg_ref[0] + agg_ref[1] + xws_ref[...]
    h = jax.nn.relu(a * dinv_ref[...] + b_ref[...])
    m = jnp.max(h, axis=1, keepdims=True)
    ex = jnp.exp(h - m)
    s = ex / jnp.sum(ex, axis=1, keepdims=True)
    s_ref[...] = s
    sb_ref[...] = s.astype(jnp.bfloat16)

    px = jnp.dot(s.T, z_ref[...], preferred_element_type=jnp.float32)
    py = jnp.dot(s.T, y_ref[...], preferred_element_type=jnp.float32)

    @pl.when(i == 0)
    def _():
        accx[...] = jnp.zeros_like(accx)
        accy[...] = jnp.zeros_like(accy)
    accx[...] += px
    accy[...] += py

    @pl.when(i == GRID - 1)
    def _():
        xnew_ref[...] = accx[...]
        syo_ref[...] = accy[...]


def _k3(agg, xws, dinv, b2, Z, Y_old):
    return pl.pallas_call(
        _k3_body,
        grid=(GRID,),
        in_specs=[
            pl.BlockSpec((SC_CORES, BN, D), lambda i: (0, i, 0)),
            pl.BlockSpec((BN, D), lambda i: (i, 0)),
            pl.BlockSpec((BN, 1), lambda i: (i, 0)),
            pl.BlockSpec((1, D), lambda i: (0, 0)),
            pl.BlockSpec((BN, D), lambda i: (i, 0)),
            pl.BlockSpec((BN, NCLS), lambda i: (i, 0)),
        ],
        out_specs=[
            pl.BlockSpec((BN, D), lambda i: (i, 0)),
            pl.BlockSpec((BN, D), lambda i: (i, 0)),
            pl.BlockSpec((D, D), lambda i: (0, 0)),
            pl.BlockSpec((D, NCLS), lambda i: (0, 0)),
        ],
        out_shape=[
            jax.ShapeDtypeStruct((N, D), jnp.float32),
            jax.ShapeDtypeStruct((N, D), jnp.bfloat16),
            jax.ShapeDtypeStruct((D, D), jnp.float32),
            jax.ShapeDtypeStruct((D, NCLS), jnp.float32),
        ],
        scratch_shapes=[
            pltpu.VMEM((D, D), jnp.float32),
            pltpu.VMEM((D, NCLS), jnp.float32),
        ],
    )(agg, xws, dinv, b2.reshape(1, D), Z, Y_old)


def _k4_body(tmp_ref, s_ref, syo_ref, anew_ref, ynp_ref, ynew_ref, acca):
    i = pl.program_id(0)
    t = tmp_ref[0] + tmp_ref[1]
    pa = jnp.dot(t.T, s_ref[...], preferred_element_type=jnp.float32)

    @pl.when(i == 0)
    def _():
        acca[...] = jnp.zeros_like(acca)
    acca[...] += pa

    @pl.when(i == GRID - 1)
    def _():
        anew_ref[...] = acca[...]
        syo = syo_ref[...]
        m = jnp.max(syo, axis=1, keepdims=True)
        ex = jnp.exp(syo - m)
        prob = ex / jnp.sum(ex, axis=1, keepdims=True)
        ynp_ref[...] = prob
        pm = jnp.max(prob, axis=1, keepdims=True)
        col = jax.lax.broadcasted_iota(jnp.int32, (D, NCLS), 1)
        big = jnp.int32(NCLS + 1)
        idx = jnp.min(jnp.where(prob == pm, col, big), axis=1, keepdims=True)
        ynew_ref[...] = jnp.where(col == idx, 1.0, 0.0).astype(jnp.float32)


def _k4(tmp, S, SYo):
    return pl.pallas_call(
        _k4_body,
        grid=(GRID,),
        in_specs=[
            pl.BlockSpec((SC_CORES, BN, D), lambda i: (0, i, 0)),
            pl.BlockSpec((BN, D), lambda i: (i, 0)),
            pl.BlockSpec((D, NCLS), lambda i: (0, 0)),
        ],
        out_specs=[
            pl.BlockSpec((D, D), lambda i: (0, 0)),
            pl.BlockSpec((D, NCLS), lambda i: (0, 0)),
            pl.BlockSpec((D, NCLS), lambda i: (0, 0)),
        ],
        out_shape=[
            jax.ShapeDtypeStruct((D, D), jnp.float32),
            jax.ShapeDtypeStruct((D, NCLS), jnp.float32),
            jax.ShapeDtypeStruct((D, NCLS), jnp.float32),
        ],
        scratch_shapes=[pltpu.VMEM((D, D), jnp.float32)],
    )(tmp, S, SYo)


_Q = np.empty((D,), np.int32)
for _m in range(D // 32):
    for _s in range(16):
        _Q[32 * _m + _s] = 32 * _m + 2 * _s
        _Q[32 * _m + 16 + _s] = 32 * _m + 2 * _s + 1
_QINV = np.argsort(_Q)
_Q2 = _Q[_Q]
_Q2INV = np.argsort(_Q2)


def _as_i32(xb):
    return lax.bitcast_convert_type(xb.reshape(N, D // 2, 2), jnp.int32)


def kernel(X_old, edge_index, edge_weight, A_old, Y_old, Z, W1, b1, W2, b2,
           use_sparse):
    del A_old, use_sparse
    eidx5 = edge_index.reshape(2, NTILES, NSTAGE, SB, B)

    q = jnp.asarray(_Q)
    qinv = jnp.asarray(_QINV)
    q2inv = jnp.asarray(_Q2INV)
    W1q = W1[:, q]
    b1q = b1[q]
    W2B = W2[q, :]
    W2A = W2B[:, q]
    b2q = b2[q]

    deg_parts = _deg_kernel(eidx5, edge_weight)
    deg2 = deg_parts.reshape(SC_CORES, N_PAD)[:, :N].T
    xws1, xb1, dinv = _k1(deg2, X_old, W1q, W1)

    agg1 = _agg_fwd(_as_i32(xb1), eidx5, edge_weight)
    xws2, xb2 = _k2(agg1, xws1, dinv, b1q, W2A, W2B)

    agg2 = _agg_fwd(_as_i32(xb2), eidx5, edge_weight)
    Sq, Sb, Xq, SYoq = _k3(agg2, xws2, dinv, b2q, Z, Y_old)

    tmp = _agg_rev(_as_i32(Sb), eidx5, edge_weight)
    Aqq, Ypq, Ynq = _k4(tmp, Sq, SYoq)

    S = Sq[:, qinv]
    X_new = Xq[qinv, :]
    A_new = Aqq[q2inv, :][:, qinv]
    Y_new_prob = Ypq[qinv, :]
    Y_new = Ynq[qinv, :]

    return (S, X_new, A_new, Y_new, Y_new_prob)


# speedup vs baseline: 2.1448x; 2.1448x over previous
import functools

import jax
import jax.numpy as jnp
from jax import lax
from jax.experimental import pallas as pl
from jax.experimental.pallas import tpu as pltpu
from jax.experimental.pallas import tpu_sc as plsc

N = 10000
E = 320000
D = 128
NCLS = 16

SC_CORES = 2
SC_SUBCORES = 16
NTILES = SC_CORES * SC_SUBCORES
EPT = E // NTILES

B = 80
NCHUNK = EPT // B
SB = 25
NSTAGE = NCHUNK // SB
NSLOT = 4
NQUAD = (SB - 1) // NSLOT

BD = 2000
NDCHUNK = EPT // BD
N_PAD = 10240
NSPAN = N_PAD // SC_SUBCORES

_mesh = plsc.VectorSubcoreMesh(
    core_axis_name="c", subcore_axis_name="s",
    num_cores=SC_CORES, num_subcores=SC_SUBCORES)


@functools.partial(
    pl.kernel,
    out_type=jax.ShapeDtypeStruct((SC_CORES * N_PAD,), jnp.float32),
    mesh=_mesh,
    compiler_params=pltpu.CompilerParams(needs_layout_passes=False),
    scratch_types=[
        pltpu.VMEM_SHARED((SC_SUBCORES, N_PAD), jnp.float32),
        pltpu.VMEM((N_PAD,), jnp.float32),
        pltpu.VMEM((SB, B), jnp.int32),
        pltpu.VMEM((BD,), jnp.float32),
        pltpu.VMEM((NSPAN,), jnp.float32),
        pltpu.VMEM((NSPAN,), jnp.float32),
    ],
)
def _deg_kernel(eidx_hbm, w_hbm, out_hbm, parts_sh, deg_v, idx_v, w_v,
                r_acc, r_op):
    cid = lax.axis_index("c")
    sid = lax.axis_index("s")
    wid = cid * SC_SUBCORES + sid

    zero16 = jnp.zeros((16,), jnp.float32)

    def z_body(i, _):
        deg_v[pl.ds(i * 16, 16)] = zero16
        return 0
    lax.fori_loop(0, N_PAD // 16, z_body, 0)

    base = pl.multiple_of(wid * EPT, 8)

    def chunk_body(ci, _):
        off = pl.multiple_of(base + ci * BD, 8)
        pltpu.sync_copy(eidx_hbm.at[1, wid, ci], idx_v)
        pltpu.sync_copy(w_hbm.at[pl.ds(off, BD)], w_v)

        def g_body(r, _):
            for g in range(B // 16):
                idx16 = idx_v[r, pl.ds(g * 16, 16)]
                w16 = w_v[pl.ds(r * B + g * 16, 16)]
                plsc.addupdate_scatter(deg_v, [idx16], w16)
            return 0
        lax.fori_loop(0, SB, g_body, 0)
        return 0
    lax.fori_loop(0, NDCHUNK, chunk_body, 0)

    pltpu.sync_copy(deg_v, parts_sh.at[sid])
    plsc.subcore_barrier()

    span = pl.multiple_of(sid * NSPAN, 8)
    pltpu.sync_copy(parts_sh.at[0, pl.ds(span, NSPAN)], r_acc)

    def red_body(t, _):
        pltpu.sync_copy(parts_sh.at[t + 1, pl.ds(span, NSPAN)], r_op)

        def add_body(i, _):
            sl = pl.ds(i * 16, 16)
            r_acc[sl] = r_acc[sl] + r_op[sl]
            return 0
        lax.fori_loop(0, NSPAN // 16, add_body, 0)
        return 0
    lax.fori_loop(0, SC_SUBCORES - 1, red_body, 0)

    oof = pl.multiple_of(cid * N_PAD + span, 8)
    pltpu.sync_copy(r_acc, out_hbm.at[pl.ds(oof, NSPAN)])


def _make_agg(gdim, sdim):
    @functools.partial(
        pl.kernel,
        out_type=jax.ShapeDtypeStruct((SC_CORES, N, D), jnp.float32),
        mesh=_mesh,
        compiler_params=pltpu.CompilerParams(needs_layout_passes=False),
        scratch_types=[
            pltpu.VMEM_SHARED((N, D), jnp.float32),
            pltpu.VMEM((SB, B), jnp.int32),
            pltpu.VMEM((SB, B), jnp.int32),
        ] + [pltpu.VMEM((B, D), jnp.float32)] * NSLOT
          + [pltpu.VMEM((B,), jnp.float32)] * NSLOT
          + [pltpu.SemaphoreType.DMA] * (3 * NSLOT),
    )
    def agg(v_hbm, eidx_hbm, w_hbm, out_hbm, acc_sh, gid_l, sid_l, *rest):
        bufs = rest[:NSLOT]
        wbufs = rest[NSLOT:2 * NSLOT]
        gsems = rest[2 * NSLOT:3 * NSLOT]
        ssems = rest[3 * NSLOT:4 * NSLOT]
        wsems = rest[4 * NSLOT:5 * NSLOT]

        cid = lax.axis_index("c")
        sid = lax.axis_index("s")
        wid = cid * SC_SUBCORES + sid
        wbase = pl.multiple_of(wid * EPT, 8)

        zero16 = jnp.zeros((16,), jnp.float32)

        def zb_body(i, _):
            for k in range(D // 16):
                bufs[0][i, pl.ds(k * 16, 16)] = zero16
            return 0
        lax.fori_loop(0, B, zb_body, 0)

        nblk = N // 80

        def z_issue(t, _):
            b = sid + t * SC_SUBCORES

            @pl.when(b < nblk)
            def _():
                pltpu.async_copy(
                    bufs[0],
                    acc_sh.at[pl.ds(pl.multiple_of(b * 80, 8), 80)],
                    gsems[0])
            return 0
        lax.fori_loop(0, 8, z_issue, 0)

        def z_wait(t, _):
            b = sid + t * SC_SUBCORES

            @pl.when(b < nblk)
            def _():
                pltpu.make_async_copy(
                    bufs[0], acc_sh.at[pl.ds(0, 80)], gsems[0]).wait()
            return 0
        lax.fori_loop(0, 8, z_wait, 0)
        plsc.subcore_barrier()

        def issue_gather(sb, c, p):
            pltpu.async_copy(v_hbm.at[gid_l.at[c]], bufs[p], gsems[p])
            woff = pl.multiple_of(wbase + (sb * SB + c) * B, 8)
            pltpu.async_copy(w_hbm.at[pl.ds(woff, B)], wbufs[p], wsems[p])

        def wait_gather(p):
            pltpu.make_async_copy(v_hbm.at[gid_l.at[0]], bufs[p],
                                  gsems[p]).wait()
            pltpu.make_async_copy(w_hbm.at[pl.ds(0, B)], wbufs[p],
                                  wsems[p]).wait()

        def issue_scatter(c, p):
            pltpu.async_copy(bufs[p], acc_sh.at[sid_l.at[c]], ssems[p],
                             add=True)

        def wait_scatter(p):
            pltpu.make_async_copy(bufs[p], acc_sh.at[sid_l.at[0]],
                                  ssems[p]).wait()

        def scale(c, p):
            rows = bufs[p]
            wv = wbufs[p]

            def g_body(g, _):
                w16 = wv[pl.ds(g * 16, 16)]
                for j in range(16):
                    e = g * 16 + j
                    wb = jnp.full((16,), w16[j], jnp.float32)
                    for k in range(D // 16):
                        sl = pl.ds(k * 16, 16)
                        rows[e, sl] = rows[e, sl] * wb
                return 0
            lax.fori_loop(0, B // 16, g_body, 0)

        def block_body(sb, _):
            pltpu.sync_copy(eidx_hbm.at[gdim, wid, sb], gid_l)
            pltpu.sync_copy(eidx_hbm.at[sdim, wid, sb], sid_l)

            issue_gather(sb, 0, 0)
            issue_gather(sb, 1, 1)

            def quad_body(t, _):
                for j in range(NSLOT):
                    c = t * NSLOT + j
                    pnext = (j + 2) % NSLOT
                    if j < 2:
                        @pl.when(t > 0)
                        def _():
                            wait_scatter(pnext)
                    else:
                        wait_scatter(pnext)
                    if j == NSLOT - 1:
                        @pl.when(t < NQUAD - 1)
                        def _():
                            issue_gather(sb, c + 2, pnext)
                    else:
                        issue_gather(sb, c + 2, pnext)
                    wait_gather(j)
                    scale(c, j)
                    issue_scatter(c, j)
                return 0
            lax.fori_loop(0, NQUAD, quad_body, 0)

            wait_scatter(2)
            wait_gather(0)
            scale(SB - 1, 0)
            issue_scatter(SB - 1, 0)
            wait_scatter(3)
            wait_scatter(0)
            return 0

        lax.fori_loop(0, NSTAGE, block_body, 0)

        plsc.subcore_barrier()

        def o_issue(t, _):
            b = sid + t * SC_SUBCORES

            @pl.when(b < nblk)
            def _():
                ro = pl.multiple_of(b * 80, 8)
                pltpu.async_copy(acc_sh.at[pl.ds(ro, 80)],
                                 out_hbm.at[cid, pl.ds(ro, 80)], gsems[0])
            return 0
        lax.fori_loop(0, 8, o_issue, 0)

        def o_wait(t, _):
            b = sid + t * SC_SUBCORES

            @pl.when(b < nblk)
            def _():
                pltpu.make_async_copy(acc_sh.at[pl.ds(0, 80)],
                                      out_hbm.at[cid, pl.ds(0, 80)],
                                      gsems[0]).wait()
            return 0
        lax.fori_loop(0, 8, o_wait, 0)

    return agg


_agg_fwd = _make_agg(0, 1)
_agg_rev = _make_agg(1, 0)


BN = 2000
GRID = N // BN


def _k1_body(degp_ref, x_ref, w1_ref, xws_ref, dinv_ref):
    deg = jnp.sum(degp_ref[...], axis=1) + 1.0
    dinv = lax.rsqrt(deg)
    xw = jnp.dot(x_ref[...], w1_ref[...], preferred_element_type=jnp.float32)
    xws_ref[...] = xw * dinv[:, None]
    dinv_ref[...] = dinv[:, None]


def _k1(deg_parts, X, W1):
    return pl.pallas_call(
        _k1_body,
        grid=(GRID,),
        in_specs=[
            pl.BlockSpec((BN, SC_CORES), lambda i: (i, 0)),
            pl.BlockSpec((BN, D), lambda i: (i, 0)),
            pl.BlockSpec((D, D), lambda i: (0, 0)),
        ],
        out_specs=[
            pl.BlockSpec((BN, D), lambda i: (i, 0)),
            pl.BlockSpec((BN, 1), lambda i: (i, 0)),
        ],
        out_shape=[
            jax.ShapeDtypeStruct((N, D), jnp.float32),
            jax.ShapeDtypeStruct((N, 1), jnp.float32),
        ],
    )(deg_parts, X, W1)


def _k2_body(agg_ref, xws_ref, dinv_ref, b_ref, w2_ref, out_ref):
    a = agg_ref[0] + agg_ref[1] + xws_ref[...]
    h = jax.nn.relu(a * dinv_ref[...] + b_ref[...])
    xw2 = jnp.dot(h, w2_ref[...], preferred_element_type=jnp.float32)
    out_ref[...] = xw2 * dinv_ref[...]


def _k2(agg, xws, dinv, b1, W2):
    return pl.pallas_call(
        _k2_body,
        grid=(GRID,),
        in_specs=[
            pl.BlockSpec((SC_CORES, BN, D), lambda i: (0, i, 0)),
            pl.BlockSpec((BN, D), lambda i: (i, 0)),
            pl.BlockSpec((BN, 1), lambda i: (i, 0)),
            pl.BlockSpec((1, D), lambda i: (0, 0)),
            pl.BlockSpec((D, D), lambda i: (0, 0)),
        ],
        out_specs=pl.BlockSpec((BN, D), lambda i: (i, 0)),
        out_shape=jax.ShapeDtypeStruct((N, D), jnp.float32),
    )(agg, xws, dinv, b1.reshape(1, D), W2)


def _k3_body(agg_ref, xws_ref, dinv_ref, b_ref, s_ref):
    a = agg_ref[0] + agg_ref[1] + xws_ref[...]
    h = jax.nn.relu(a * dinv_ref[...] + b_ref[...])
    m = jnp.max(h, axis=1, keepdims=True)
    ex = jnp.exp(h - m)
    s_ref[...] = ex / jnp.sum(ex, axis=1, keepdims=True)


def _k3(agg, xws, dinv, b2):
    return pl.pallas_call(
        _k3_body,
        grid=(GRID,),
        in_specs=[
            pl.BlockSpec((SC_CORES, BN, D), lambda i: (0, i, 0)),
            pl.BlockSpec((BN, D), lambda i: (i, 0)),
            pl.BlockSpec((BN, 1), lambda i: (i, 0)),
            pl.BlockSpec((1, D), lambda i: (0, 0)),
        ],
        out_specs=pl.BlockSpec((BN, D), lambda i: (i, 0)),
        out_shape=jax.ShapeDtypeStruct((N, D), jnp.float32),
    )(agg, xws, dinv, b2.reshape(1, D))


def _k3b_body(s_ref, z_ref, y_ref, xnew_ref, syo_ref, accx, accy):
    i = pl.program_id(0)
    s = s_ref[...]
    px = jnp.dot(s.T, z_ref[...], preferred_element_type=jnp.float32)
    py = jnp.dot(s.T, y_ref[...], preferred_element_type=jnp.float32)

    @pl.when(i == 0)
    def _():
        accx[...] = jnp.zeros_like(accx)
        accy[...] = jnp.zeros_like(accy)
    accx[...] += px
    accy[...] += py

    @pl.when(i == GRID - 1)
    def _():
        xnew_ref[...] = accx[...]
        syo_ref[...] = accy[...]


def _k3b(S, Z, Y_old):
    return pl.pallas_call(
        _k3b_body,
        grid=(GRID,),
        in_specs=[
            pl.BlockSpec((BN, D), lambda i: (i, 0)),
            pl.BlockSpec((BN, D), lambda i: (i, 0)),
            pl.BlockSpec((BN, NCLS), lambda i: (i, 0)),
        ],
        out_specs=[
            pl.BlockSpec((D, D), lambda i: (0, 0)),
            pl.BlockSpec((D, NCLS), lambda i: (0, 0)),
        ],
        out_shape=[
            jax.ShapeDtypeStruct((D, D), jnp.float32),
            jax.ShapeDtypeStruct((D, NCLS), jnp.float32),
        ],
        scratch_shapes=[
            pltpu.VMEM((D, D), jnp.float32),
            pltpu.VMEM((D, NCLS), jnp.float32),
        ],
    )(S, Z, Y_old)


def _k4_body(tmp_ref, s_ref, syo_ref, anew_ref, ynp_ref, ynew_ref, acca):
    i = pl.program_id(0)
    t = tmp_ref[0] + tmp_ref[1]
    pa = jnp.dot(t.T, s_ref[...], preferred_element_type=jnp.float32)

    @pl.when(i == 0)
    def _():
        acca[...] = jnp.zeros_like(acca)
    acca[...] += pa

    @pl.when(i == GRID - 1)
    def _():
        anew_ref[...] = acca[...]
        syo = syo_ref[...]
        m = jnp.max(syo, axis=1, keepdims=True)
        ex = jnp.exp(syo - m)
        prob = ex / jnp.sum(ex, axis=1, keepdims=True)
        ynp_ref[...] = prob
        pm = jnp.max(prob, axis=1, keepdims=True)
        col = jax.lax.broadcasted_iota(jnp.int32, (D, NCLS), 1)
        big = jnp.int32(NCLS + 1)
        idx = jnp.min(jnp.where(prob == pm, col, big), axis=1, keepdims=True)
        ynew_ref[...] = jnp.where(col == idx, 1.0, 0.0).astype(jnp.float32)


def _k4(tmp, S, SYo):
    return pl.pallas_call(
        _k4_body,
        grid=(GRID,),
        in_specs=[
            pl.BlockSpec((SC_CORES, BN, D), lambda i: (0, i, 0)),
            pl.BlockSpec((BN, D), lambda i: (i, 0)),
            pl.BlockSpec((D, NCLS), lambda i: (0, 0)),
        ],
        out_specs=[
            pl.BlockSpec((D, D), lambda i: (0, 0)),
            pl.BlockSpec((D, NCLS), lambda i: (0, 0)),
            pl.BlockSpec((D, NCLS), lambda i: (0, 0)),
        ],
        out_shape=[
            jax.ShapeDtypeStruct((D, D), jnp.float32),
            jax.ShapeDtypeStruct((D, NCLS), jnp.float32),
            jax.ShapeDtypeStruct((D, NCLS), jnp.float32),
        ],
        scratch_shapes=[pltpu.VMEM((D, D), jnp.float32)],
    )(tmp, S, SYo)


def kernel(X_old, edge_index, edge_weight, A_old, Y_old, Z, W1, b1, W2, b2,
           use_sparse):
    del A_old, use_sparse
    eidx5 = edge_index.reshape(2, NTILES, NSTAGE, SB, B)

    deg_parts = _deg_kernel(eidx5, edge_weight)
    deg2 = deg_parts.reshape(SC_CORES, N_PAD)[:, :N].T
    xws1, dinv = _k1(deg2, X_old, W1)

    agg1 = _agg_fwd(xws1, eidx5, edge_weight)
    xws2 = _k2(agg1, xws1, dinv, b1, W2)

    agg2 = _agg_fwd(xws2, eidx5, edge_weight)
    S = _k3(agg2, xws2, dinv, b2)

    tmp = _agg_rev(S, eidx5, edge_weight)
    X_new, SYo = _k3b(S, Z, Y_old)
    A_new, Y_new_prob, Y_new = _k4(tmp, S, SYo)

    return (S, X_new, A_new, Y_new, Y_new_prob)
